# baseline (speedup 1.0000x reference)
import jax
import jax.numpy as jnp
from jax.experimental import pallas as pl
from jax.experimental.pallas import tpu as pltpu

N = 1024
H = 8
C = 16
FEAT = H * C
TJ = 1024
NEG = -1e30


def _fused_kernel(x_ref, adj_ref, wl1_ref, bl1_ref, wr1_ref, br1_ref,
                  att1_ref, bias1_ref, wl2_ref, bl2_ref, wr2_ref, br2_ref,
                  att2_ref, bias2_ref, out_ref,
                  xl_s, xltb_s, xrb_s, ut_s, vt_s, h1_s):
    l = pl.program_id(0)
    j = pl.program_id(1)
    j0 = j * TJ

    def _proj(x, wl_ref, bl_ref, wr_ref, br_ref, att_ref):
        xl = jnp.dot(x, wl_ref[...], preferred_element_type=jnp.float32) \
            + bl_ref[...]
        xr = jnp.dot(x, wr_ref[...], preferred_element_type=jnp.float32) \
            + br_ref[...]
        xlt = xl.T
        xrt = xr.T
        ut_rows = []
        vt_rows = []
        for h in range(H):
            u_row = jnp.zeros((1, N), jnp.float32)
            v_row = jnp.zeros((1, N), jnp.float32)
            for c in range(C):
                f = h * C + c
                a6 = 0.6 * att_ref[h, c]
                u_row = u_row + a6 * xlt[f:f + 1, :]
                v_row = v_row + a6 * xrt[f:f + 1, :]
            ut_rows.append(u_row)
            vt_rows.append(v_row)
        xl_s[...] = xl
        xltb_s[...] = xlt.astype(jnp.bfloat16)
        xrb_s[...] = xr.astype(jnp.bfloat16)
        ut_s[...] = jnp.concatenate(ut_rows, axis=0)
        vt_s[...] = jnp.concatenate(vt_rows, axis=0)

    @pl.when(jnp.logical_and(l == 0, j == 0))
    def _proj1():
        _proj(x_ref[...], wl1_ref, bl1_ref, wr1_ref, br1_ref, att1_ref)

    @pl.when(jnp.logical_and(l == 1, j == 0))
    def _proj2():
        _proj(h1_s[...], wl2_ref, bl2_ref, wr2_ref, br2_ref, att2_ref)

    adj_t = adj_ref[...].T
    row_j = jax.lax.broadcasted_iota(jnp.int32, (TJ, N), 0) + j0
    col_i = jax.lax.broadcasted_iota(jnp.int32, (TJ, N), 1)
    valid = jnp.logical_or(row_j == col_i, adj_t != 0)

    v_blk = vt_s[:, pl.ds(j0, TJ)].T
    outs = []
    for h in range(H):
        acc_abs = None
        for c in range(C):
            f = h * C + c
            a1 = 0.4 * att1_ref[h, c]
            a2 = 0.4 * att2_ref[h, c]
            a4 = jnp.where(l == 0, a1, a2).astype(jnp.bfloat16)
            s = xrb_s[pl.ds(j0, TJ), f:f + 1] + xltb_s[f:f + 1, :]
            t = a4 * jnp.abs(s)
            acc_abs = t if acc_abs is None else acc_abs + t
        acc_abs = jnp.where(valid, acc_abs, jnp.bfloat16(NEG))
        acc = (v_blk[:, h:h + 1] + ut_s[h:h + 1, :]
               + acc_abs.astype(jnp.float32))
        m = jnp.max(acc, axis=1, keepdims=True)
        p = jnp.exp(acc - m)
        den = jnp.sum(p, axis=1, keepdims=True) + 1e-16
        agg = jnp.dot(p, xl_s[:, h * C:(h + 1) * C],
                      preferred_element_type=jnp.float32)
        outs.append(agg / den)
    bias = jnp.where(l == 0, bias1_ref[...], bias2_ref[...])
    out = jnp.concatenate(outs, axis=1) + bias

    @pl.when(l == 0)
    def _store_h1():
        h1_s[pl.ds(j0, TJ), :] = out
        out_ref[...] = out

    @pl.when(l == 1)
    def _store_out():
        out_ref[...] = jnp.where(
            out > 0, out, jnp.exp(jnp.minimum(out, 0.0)) - 1.0)


def kernel(input, adj, Wl1, bl1, Wr1, br1, att1, bias1,
           Wl2, bl2, Wr2, br2, att2, bias2):
    b, n, ic, nf = input.shape
    x = input.reshape(n, ic * nf)
    adj32 = adj.astype(jnp.int32)
    full = pl.BlockSpec((FEAT, FEAT), lambda l, j: (0, 0))
    brow = pl.BlockSpec((1, FEAT), lambda l, j: (0, 0))
    smem = pl.BlockSpec(memory_space=pltpu.SMEM)

    h2 = pl.pallas_call(
        _fused_kernel,
        grid=(2, N // TJ),
        in_specs=[
            pl.BlockSpec((N, FEAT), lambda l, j: (0, 0)),
            pl.BlockSpec((N, TJ), lambda l, j: (0, j)),
            full, brow, full, brow, smem, brow,
            full, brow, full, brow, smem, brow,
        ],
        out_specs=pl.BlockSpec((TJ, FEAT), lambda l, j: (j, 0)),
        out_shape=jax.ShapeDtypeStruct((N, FEAT), jnp.float32),
        scratch_shapes=[
            pltpu.VMEM((N, FEAT), jnp.float32),
            pltpu.VMEM((FEAT, N), jnp.bfloat16),
            pltpu.VMEM((N, FEAT), jnp.bfloat16),
            pltpu.VMEM((H, N), jnp.float32),
            pltpu.VMEM((H, N), jnp.float32),
            pltpu.VMEM((N, FEAT), jnp.float32),
        ],
        compiler_params=pltpu.CompilerParams(
            dimension_semantics=("arbitrary", "arbitrary")),
    )(x, adj32,
      Wl1, bl1.reshape(1, FEAT), Wr1, br1.reshape(1, FEAT), att1,
      bias1.reshape(1, FEAT),
      Wl2, bl2.reshape(1, FEAT), Wr2, br2.reshape(1, FEAT), att2,
      bias2.reshape(1, FEAT))
    return h2.reshape(b, n, H * C)


# speedup vs baseline: 1.3188x; 1.3188x over previous
import jax
import jax.numpy as jnp
from jax.experimental import pallas as pl
from jax.experimental.pallas import tpu as pltpu

N = 1024
H = 8
C = 16
FEAT = H * C
TJ = 512
NEG = -1e30


def _fused_kernel(x_ref, adj_ref, wl1_ref, bl1_ref, wr1_ref, br1_ref,
                  att1_ref, bias1_ref, wl2_ref, bl2_ref, wr2_ref, br2_ref,
                  att2_ref, bias2_ref, out_ref,
                  xl_s, xltb_s, xrb_s, ut_s, vt_s, h1_s):
    l = pl.program_id(0)
    j = pl.program_id(1)
    j0 = j * TJ

    def _proj(x, wl_ref, bl_ref, wr_ref, br_ref, att_ref):
        xl = jnp.dot(x, wl_ref[...], preferred_element_type=jnp.float32) \
            + bl_ref[...]
        xr = jnp.dot(x, wr_ref[...], preferred_element_type=jnp.float32) \
            + br_ref[...]
        xlt = xl.T
        xrt = xr.T
        ut_rows = []
        vt_rows = []
        for h in range(H):
            u_row = jnp.zeros((1, N), jnp.float32)
            v_row = jnp.zeros((1, N), jnp.float32)
            for c in range(C):
                f = h * C + c
                a6 = 0.6 * att_ref[h, c]
                u_row = u_row + a6 * xlt[f:f + 1, :]
                v_row = v_row + a6 * xrt[f:f + 1, :]
            ut_rows.append(u_row)
            vt_rows.append(v_row)
        xl_s[...] = xl
        xltb_s[...] = xlt.astype(jnp.bfloat16)
        xrb_s[...] = xr.astype(jnp.bfloat16)
        ut_s[...] = jnp.concatenate(ut_rows, axis=0).astype(jnp.bfloat16)
        vt_s[...] = jnp.concatenate(vt_rows, axis=0).astype(jnp.bfloat16)

    @pl.when(jnp.logical_and(l == 0, j == 0))
    def _proj1():
        _proj(x_ref[...], wl1_ref, bl1_ref, wr1_ref, br1_ref, att1_ref)

    @pl.when(jnp.logical_and(l == 1, j == 0))
    def _proj2():
        _proj(h1_s[...], wl2_ref, bl2_ref, wr2_ref, br2_ref, att2_ref)

    adj_t = adj_ref[...].T
    row_j = jax.lax.broadcasted_iota(jnp.int32, (TJ, N), 0) + j0
    col_i = jax.lax.broadcasted_iota(jnp.int32, (TJ, N), 1)
    valid = jnp.logical_or(row_j == col_i, adj_t != 0)

    v_blk = vt_s[:, pl.ds(j0, TJ)].T
    outs = []
    for h in range(H):
        acc_abs = None
        for c in range(C):
            f = h * C + c
            a1 = 0.4 * att1_ref[h, c]
            a2 = 0.4 * att2_ref[h, c]
            a4 = jnp.where(l == 0, a1, a2).astype(jnp.bfloat16)
            s = xrb_s[pl.ds(j0, TJ), f:f + 1] + xltb_s[f:f + 1, :]
            t = a4 * jnp.abs(s)
            acc_abs = t if acc_abs is None else acc_abs + t
        acc_bf = acc_abs + (v_blk[:, h:h + 1] + ut_s[h:h + 1, :])
        acc_bf = jnp.where(valid, acc_bf, jnp.bfloat16(NEG))
        acc = acc_bf.astype(jnp.float32)
        m = jnp.max(acc, axis=1, keepdims=True)
        p = jnp.exp(acc - m)
        den = jnp.sum(p, axis=1, keepdims=True) + 1e-16
        agg = jnp.dot(p, xl_s[:, h * C:(h + 1) * C],
                      preferred_element_type=jnp.float32)
        outs.append(agg / den)
    bias = jnp.where(l == 0, bias1_ref[...], bias2_ref[...])
    out = jnp.concatenate(outs, axis=1) + bias

    @pl.when(l == 0)
    def _store_h1():
        h1_s[pl.ds(j0, TJ), :] = out
        out_ref[...] = out

    @pl.when(l == 1)
    def _store_out():
        out_ref[...] = jnp.where(
            out > 0, out, jnp.exp(jnp.minimum(out, 0.0)) - 1.0)


def kernel(input, adj, Wl1, bl1, Wr1, br1, att1, bias1,
           Wl2, bl2, Wr2, br2, att2, bias2):
    b, n, ic, nf = input.shape
    x = input.reshape(n, ic * nf)
    adj32 = adj.astype(jnp.int32)
    full = pl.BlockSpec((FEAT, FEAT), lambda l, j: (0, 0))
    brow = pl.BlockSpec((1, FEAT), lambda l, j: (0, 0))
    smem = pl.BlockSpec(memory_space=pltpu.SMEM)

    h2 = pl.pallas_call(
        _fused_kernel,
        grid=(2, N // TJ),
        in_specs=[
            pl.BlockSpec((N, FEAT), lambda l, j: (0, 0)),
            pl.BlockSpec((N, TJ), lambda l, j: (0, j)),
            full, brow, full, brow, smem, brow,
            full, brow, full, brow, smem, brow,
        ],
        out_specs=pl.BlockSpec((TJ, FEAT), lambda l, j: (j, 0)),
        out_shape=jax.ShapeDtypeStruct((N, FEAT), jnp.float32),
        scratch_shapes=[
            pltpu.VMEM((N, FEAT), jnp.float32),
            pltpu.VMEM((FEAT, N), jnp.bfloat16),
            pltpu.VMEM((N, FEAT), jnp.bfloat16),
            pltpu.VMEM((H, N), jnp.bfloat16),
            pltpu.VMEM((H, N), jnp.bfloat16),
            pltpu.VMEM((N, FEAT), jnp.float32),
        ],
        compiler_params=pltpu.CompilerParams(
            dimension_semantics=("arbitrary", "arbitrary")),
    )(x, adj32,
      Wl1, bl1.reshape(1, FEAT), Wr1, br1.reshape(1, FEAT), att1,
      bias1.reshape(1, FEAT),
      Wl2, bl2.reshape(1, FEAT), Wr2, br2.reshape(1, FEAT), att2,
      bias2.reshape(1, FEAT))
    return h2.reshape(b, n, H * C)
